# baseline (speedup 1.0000x reference)
import functools

import jax
import jax.numpy as jnp
from jax import lax
from jax.experimental import pallas as pl
from jax.experimental.pallas import tpu as pltpu
from jax.experimental.pallas import tpu_sc as plsc

_NW = 32


def _pack_kernel(V, D, NC, n_blocks, tail,
                 tt_hbm, tail_hbm, t2_hbm, src_v, dst_v, tail_v,
                 rsem0, rsem1, wsem0, wsem1):
    wid = lax.axis_index("s") * NC + lax.axis_index("c")
    rsems = (rsem0, rsem1)
    wsems = (wsem0, wsem1)
    iota = lax.iota(jnp.int32, 16)
    fvecs = [iota + fg * 16 for fg in range(4)]

    my_n = (n_blocks - wid + _NW - 1) // _NW

    def src_slice(k, b):
        blk = k * _NW + wid
        return tt_hbm.at[:, pl.ds(blk * 128, 128)], src_v.at[b]

    def dst_slice(k, b):
        blk = k * _NW + wid
        return dst_v.at[b], t2_hbm.at[pl.ds(blk * 64, 64)]

    def fire_read(k, b):
        pltpu.async_copy(*src_slice(k, b), rsems[b])

    def drain_read(k, b):
        pltpu.make_async_copy(*src_slice(k, b), rsems[b]).wait()

    def fire_write(k, b):
        pltpu.async_copy(*dst_slice(k, b), wsems[b])

    def drain_write(k, b):
        pltpu.make_async_copy(*dst_slice(k, b), wsems[b]).wait()

    def shuffle_into(src_ref, dst_ref, npairs):
        def body(p, carry):
            for half in range(2):
                c = jnp.full((16,), 0, dtype=jnp.int32) + (2 * p + half)
                for fg in range(4):
                    vals = plsc.load_gather(src_ref, [fvecs[fg], c])
                    dst_ref[p, pl.ds(half * 64 + fg * 16, 16)] = vals
            return carry
        lax.fori_loop(0, npairs, body, 0)

    def chunk(k, b, guard):
        if guard:
            @pl.when(k < my_n)
            def _():
                chunk(k, b, False)
            return
        drain_read(k, b)
        @pl.when(k >= 2)
        def _():
            drain_write(k - 2, b)
        shuffle_into(src_v.at[b], dst_v.at[b], 64)
        fire_write(k, b)
        @pl.when(k + 2 < my_n)
        def _():
            fire_read(k + 2, b)

    fire_read(0, 0)
    @pl.when(my_n >= 2)
    def _():
        fire_read(1, 1)

    def pairstep(g, carry):
        chunk(2 * g, 0, True)
        chunk(2 * g + 1, 1, True)
        return carry

    lax.fori_loop(0, (my_n + 1) // 2, pairstep, 0)

    last = my_n - 1
    @pl.when(lax.rem(last, 2) == 0)
    def _():
        @pl.when(my_n >= 2)
        def _():
            drain_write(my_n - 2, 1)
        drain_write(last, 0)
    @pl.when(lax.rem(last, 2) == 1)
    def _():
        drain_write(my_n - 2, 0)
        drain_write(last, 1)

    if tail:
        @pl.when(wid == 0)
        def _():
            pltpu.sync_copy(tail_hbm, tail_v)
            shuffle_into(tail_v, dst_v.at[0], tail // 2)
            pltpu.sync_copy(
                dst_v.at[0].at[pl.ds(0, tail // 2)],
                t2_hbm.at[pl.ds((V - tail) // 2, tail // 2)],
            )


def _gather_kernel(L, D, B, NC,
                   xp_hbm, t2_hbm, out_hbm,
                   xp_v, pidx_v, cidx_v, rows_v, slab_v,
                   gsem0, gsem1, wsem0, wsem1):
    wid = lax.axis_index("s") * NC + lax.axis_index("c")
    b0 = wid * 128
    gsems = (gsem0, gsem1)
    wsems = (wsem0, wsem1)
    iota = lax.iota(jnp.int32, 16)

    pltpu.sync_copy(xp_hbm.at[:, pl.ds(b0, 128)], xp_v)

    def prep(l, b):
        for g in range(8):
            r = xp_v[l, pl.ds(g * 16, 16)]
            pidx_v[b, pl.ds(g * 16, 16)] = r // 2
            cidx_v[b, pl.ds(g * 16, 16)] = (r & 1) * 64

    def fire_gather(b):
        pltpu.async_copy(t2_hbm.at[pidx_v.at[b]], rows_v.at[b], gsems[b])

    def drain_gather(b):
        pltpu.make_async_copy(
            t2_hbm.at[pidx_v.at[b]], rows_v.at[b], gsems[b]).wait()

    def out_slice(l, b):
        return slab_v.at[b], out_hbm.at[l, :, pl.ds(b0, 128)]

    def fire_write(l, b):
        pltpu.async_copy(*out_slice(l, b), wsems[b])

    def drain_write(l, b):
        pltpu.make_async_copy(*out_slice(l, b), wsems[b]).wait()

    def select(b):
        for g in range(8):
            jvec = iota + g * 16
            cvec0 = cidx_v[b, pl.ds(g * 16, 16)]

            def fbody(f, cvec):
                slab_v[b, f, pl.ds(g * 16, 16)] = plsc.load_gather(
                    rows_v.at[b], [jvec, cvec])
                return cvec + 1
            lax.fori_loop(0, D, fbody, cvec0, unroll=4)

    def step(l, b, first, last):
        if not first:
            drain_write(l - 2, b)
        drain_gather(b)
        select(b)
        fire_write(l, b)
        if not last:
            prep(l + 2, b)
            fire_gather(b)

    prep(0, 0)
    fire_gather(0)
    prep(1, 1)
    fire_gather(1)

    step(0, 0, True, False)
    step(1, 1, True, False)

    def pairstep(g, carry):
        step(2 * g, 0, False, False)
        step(2 * g + 1, 1, False, False)
        return carry

    lax.fori_loop(1, L // 2 - 1, pairstep, 0)

    step(L - 2, 0, False, True)
    step(L - 1, 1, False, True)
    drain_write(L - 2, 0)
    drain_write(L - 1, 1)


def kernel(x, table):
    B, L = x.shape
    V, D = table.shape
    assert D == 64 and V % 2 == 0 and B % (_NW * 128) == 0 and L % 4 == 0

    info = plsc.get_sparse_core_info()
    NC, NS = info.num_cores, info.num_subcores
    assert NC * NS == _NW

    idx = x.astype(jnp.int32)
    xp = idx.T
    tt = table.T
    n_blocks = V // 128
    tail = V - n_blocks * 128
    t_tail = tt[:, n_blocks * 128:] if tail else jnp.zeros((D, 2), jnp.float32)

    mesh = plsc.VectorSubcoreMesh(core_axis_name="c", subcore_axis_name="s")

    pack = pl.kernel(
        functools.partial(_pack_kernel, V, D, NC, n_blocks, tail),
        mesh=mesh,
        out_type=jax.ShapeDtypeStruct((V // 2, 2 * D), jnp.float32),
        scratch_types=[
            pltpu.VMEM((2, D, 128), jnp.float32),
            pltpu.VMEM((2, 64, 2 * D), jnp.float32),
            pltpu.VMEM((D, max(tail, 2)), jnp.float32),
            pltpu.SemaphoreType.DMA,
            pltpu.SemaphoreType.DMA,
            pltpu.SemaphoreType.DMA,
            pltpu.SemaphoreType.DMA,
        ],
        compiler_params=pltpu.CompilerParams(
            use_tc_tiling_on_sc=True, needs_layout_passes=False),
    )
    t2 = pack(tt, t_tail)

    gather = pl.kernel(
        functools.partial(_gather_kernel, L, D, B, NC),
        mesh=mesh,
        out_type=jax.ShapeDtypeStruct((L, D, B), jnp.float32),
        scratch_types=[
            pltpu.VMEM((L, 128), jnp.int32),
            pltpu.VMEM((2, 128), jnp.int32),
            pltpu.VMEM((2, 128), jnp.int32),
            pltpu.VMEM((2, 128, 2 * D), jnp.float32),
            pltpu.VMEM((2, D, 128), jnp.float32),
            pltpu.SemaphoreType.DMA,
            pltpu.SemaphoreType.DMA,
            pltpu.SemaphoreType.DMA,
            pltpu.SemaphoreType.DMA,
        ],
        compiler_params=pltpu.CompilerParams(
            use_tc_tiling_on_sc=True, needs_layout_passes=False),
    )
    out_phys = gather(xp, t2)
    return jnp.transpose(out_phys, (2, 0, 1))
